# baseline (speedup 1.0000x reference)
import functools

import jax
import jax.numpy as jnp
import numpy as np
from jax import lax
from jax.experimental import pallas as pl
from jax.experimental.pallas import tpu as pltpu
from jax.experimental.pallas import tpu_sc as plsc

_MASK_FRAC = 0.5
_S = 16
_H = 224
_SCALE = _H // _S
_NPIX = _H * _H
_NP = 768
_NC = 2
_NS = 16
_NW = _NC * _NS
_PPW = _NP // _NW
_VL = 16
_VPR = _H // _VL


def _host_tables():
    s = (np.arange(_H) + 0.5) / _SCALE - 0.5
    f = np.floor(s)
    w = (s - f).astype(np.float32)
    i0 = np.clip(f, 0, _S - 1).astype(np.int32)
    i1 = np.clip(f + 1, 0, _S - 1).astype(np.int32)
    return i0, i1, (1.0 - w), w


def _sc_body(x_hbm, mask_hbm, i0_hbm, i1_hbm, w0_hbm, w1_hbm, out_hbm,
             xb, mh, mv, i0v, i1v, w0v, w1v):
    wid = lax.axis_index("s") * _NC + lax.axis_index("c")

    pltpu.sync_copy(i0_hbm, i0v)
    pltpu.sync_copy(i1_hbm, i1v)
    pltpu.sync_copy(w0_hbm, w0v)
    pltpu.sync_copy(w1_hbm, w1v)

    def plane_body(k, carry):
        p = wid * _PPW + k
        pltpu.sync_copy(mask_hbm.at[p], mv)
        pltpu.sync_copy(x_hbm.at[p], xb)

        for r in range(_S):
            for v in range(_VPR):
                sl = pl.ds(v * _VL, _VL)
                g0 = plsc.load_gather(mv, [i0v[sl] + r * _S])
                g1 = plsc.load_gather(mv, [i1v[sl] + r * _S])
                mh[pl.ds(r * _H + v * _VL, _VL)] = g0 * w0v[sl] + g1 * w1v[sl]

        zero = jnp.zeros((_VL,), jnp.float32)

        def apply_const_rows(row0, nrows, mrow_base):
            bits = [mh[pl.ds(mrow_base + v * _VL, _VL)] for v in range(_VPR)]

            def row_body(i, c):
                base = (row0 + i) * _H
                for v in range(_VPR):
                    sl = pl.ds(base + v * _VL, _VL)
                    xv = xb[sl]
                    xb[sl] = jnp.where(bits[v] < _MASK_FRAC, zero, xv)
                return c
            lax.fori_loop(0, nrows, row_body, 0)

        apply_const_rows(0, _SCALE // 2, 0)

        for run in range(_S - 1):
            m0s = [mh[pl.ds(run * _H + v * _VL, _VL)] for v in range(_VPR)]
            dvs = [mh[pl.ds((run + 1) * _H + v * _VL, _VL)] - m0s[v]
                   for v in range(_VPR)]
            row0 = _SCALE // 2 + run * _SCALE

            def row_body(i, c):
                wy = (i.astype(jnp.float32) + 0.5) * (1.0 / _SCALE)
                wyv = jnp.full((_VL,), 0.0, jnp.float32) + wy
                base = (row0 + i) * _H
                for v in range(_VPR):
                    sl = pl.ds(base + v * _VL, _VL)
                    xv = xb[sl]
                    m = m0s[v] + wyv * dvs[v]
                    xb[sl] = jnp.where(m < _MASK_FRAC, zero, xv)
                return c
            lax.fori_loop(0, _SCALE, row_body, 0)

        apply_const_rows(_H - _SCALE // 2, _SCALE // 2, (_S - 1) * _H)

        pltpu.sync_copy(xb, out_hbm.at[p])
        return carry

    lax.fori_loop(0, _PPW, plane_body, 0)


@jax.jit
def _run(x, mask):
    B, C, H, W = x.shape
    xp = x.reshape(_NP, _NPIX)
    mp = mask.reshape(_NP, _S * _S)
    i0, i1, w0, w1 = _host_tables()

    mesh = plsc.VectorSubcoreMesh(core_axis_name="c", subcore_axis_name="s",
                                  num_cores=_NC, num_subcores=_NS)
    fn = functools.partial(
        pl.kernel,
        out_type=jax.ShapeDtypeStruct((_NP, _NPIX), jnp.float32),
        mesh=mesh,
        compiler_params=pltpu.CompilerParams(needs_layout_passes=False),
        scratch_types=[
            pltpu.VMEM((_NPIX,), jnp.float32),
            pltpu.VMEM((_S * _H,), jnp.float32),
            pltpu.VMEM((_S * _S,), jnp.float32),
            pltpu.VMEM((_H,), jnp.int32),
            pltpu.VMEM((_H,), jnp.int32),
            pltpu.VMEM((_H,), jnp.float32),
            pltpu.VMEM((_H,), jnp.float32),
        ],
    )(_sc_body)
    out = fn(xp, mp, jnp.asarray(i0), jnp.asarray(i1),
             jnp.asarray(w0), jnp.asarray(w1))
    return out.reshape(B, C, H, W)


def kernel(x, mask):
    return _run(x, mask)


# speedup vs baseline: 2.1657x; 2.1657x over previous
import functools

import jax
import jax.numpy as jnp
import numpy as np
from jax import lax
from jax.experimental import pallas as pl
from jax.experimental.pallas import tpu as pltpu
from jax.experimental.pallas import tpu_sc as plsc

_MASK_FRAC = 0.5
_S = 16
_H = 224
_SCALE = _H // _S
_NPIX = _H * _H
_NP = 768
_NC = 2
_NS = 16
_NW = _NC * _NS
_PPW = _NP // _NW
_VL = 16
_VPR = _H // _VL


def _host_tables():
    s = (np.arange(_H) + 0.5) / _SCALE - 0.5
    f = np.floor(s)
    w = (s - f).astype(np.float32)
    i0 = np.clip(f, 0, _S - 1).astype(np.int32)
    i1 = np.clip(f + 1, 0, _S - 1).astype(np.int32)
    return i0, i1, (1.0 - w), w


def _sc_body(x_hbm, mask_hbm, i0_hbm, i1_hbm, w0_hbm, w1_hbm, out_hbm,
             xb, mh, mv, i0v, i1v, w0v, w1v):
    wid = lax.axis_index("s") * _NC + lax.axis_index("c")

    pltpu.sync_copy(i0_hbm, i0v)
    pltpu.sync_copy(i1_hbm, i1v)
    pltpu.sync_copy(w0_hbm, w0v)
    pltpu.sync_copy(w1_hbm, w1v)

    def plane_body(k, carry):
        p = wid * _PPW + k
        pltpu.sync_copy(mask_hbm.at[p], mv)
        pltpu.sync_copy(x_hbm.at[p], xb)

        for r in range(_S):
            for v in range(_VPR):
                sl = pl.ds(v * _VL, _VL)
                g0 = plsc.load_gather(mv, [i0v[sl] + r * _S])
                g1 = plsc.load_gather(mv, [i1v[sl] + r * _S])
                mh[pl.ds(r * _H + v * _VL, _VL)] = g0 * w0v[sl] + g1 * w1v[sl]

        zero = jnp.zeros((_VL,), jnp.float32)

        def apply_const_rows(row0, nrows, mrow_base):
            bits = [mh[pl.ds(mrow_base + v * _VL, _VL)] for v in range(_VPR)]

            def row_body(i, c):
                r = row0 + i
                for v in range(_VPR):
                    sl = pl.ds(v * _VL, _VL)
                    xv = xb[r, sl]
                    xb[r, sl] = jnp.where(bits[v] < _MASK_FRAC, zero, xv)
                return c
            lax.fori_loop(0, nrows, row_body, 0)

        apply_const_rows(0, _SCALE // 2, 0)

        for run in range(_S - 1):
            m0s = [mh[pl.ds(run * _H + v * _VL, _VL)] for v in range(_VPR)]
            dvs = [mh[pl.ds((run + 1) * _H + v * _VL, _VL)] - m0s[v]
                   for v in range(_VPR)]
            row0 = _SCALE // 2 + run * _SCALE

            def row_body(i, c):
                wy = (i.astype(jnp.float32) + 0.5) * (1.0 / _SCALE)
                wyv = jnp.full((_VL,), 0.0, jnp.float32) + wy
                r = row0 + i
                for v in range(_VPR):
                    sl = pl.ds(v * _VL, _VL)
                    xv = xb[r, sl]
                    m = m0s[v] + wyv * dvs[v]
                    xb[r, sl] = jnp.where(m < _MASK_FRAC, zero, xv)
                return c
            lax.fori_loop(0, _SCALE, row_body, 0)

        apply_const_rows(_H - _SCALE // 2, _SCALE // 2, (_S - 1) * _H)

        pltpu.sync_copy(xb, out_hbm.at[p])
        return carry

    lax.fori_loop(0, _PPW, plane_body, 0)


@jax.jit
def _run(x, mask):
    B, C, H, W = x.shape
    xp = x.reshape(_NP, _H, _H)
    mp = mask.reshape(_NP, _S * _S)
    i0, i1, w0, w1 = _host_tables()

    mesh = plsc.VectorSubcoreMesh(core_axis_name="c", subcore_axis_name="s",
                                  num_cores=_NC, num_subcores=_NS)
    fn = functools.partial(
        pl.kernel,
        out_type=jax.ShapeDtypeStruct((_NP, _H, _H), jnp.float32),
        mesh=mesh,
        compiler_params=pltpu.CompilerParams(needs_layout_passes=False),
        scratch_types=[
            pltpu.VMEM((_H, _H), jnp.float32),
            pltpu.VMEM((_S * _H,), jnp.float32),
            pltpu.VMEM((_S * _S,), jnp.float32),
            pltpu.VMEM((_H,), jnp.int32),
            pltpu.VMEM((_H,), jnp.int32),
            pltpu.VMEM((_H,), jnp.float32),
            pltpu.VMEM((_H,), jnp.float32),
        ],
    )(_sc_body)
    out = fn(xp, mp, jnp.asarray(i0), jnp.asarray(i1),
             jnp.asarray(w0), jnp.asarray(w1))
    return out.reshape(B, C, H, W)


def kernel(x, mask):
    return _run(x, mask)


# speedup vs baseline: 3.4462x; 1.5913x over previous
import functools

import jax
import jax.numpy as jnp
import numpy as np
from jax import lax
from jax.experimental import pallas as pl
from jax.experimental.pallas import tpu as pltpu
from jax.experimental.pallas import tpu_sc as plsc

_MASK_FRAC = 0.5
_S = 16
_H = 224
_SCALE = _H // _S
_NP = 768
_NC = 2
_NS = 16
_NW = _NC * _NS
_PPW = _NP // _NW
_VL = 16
_VPR = _H // _VL
_CH = 56

_SEGS = {
    0: [("c", 0, 7, 0, 0), ("l", 7, 14, 0, 0), ("l", 21, 14, 1, 0),
        ("l", 35, 14, 2, 0), ("l", 49, 7, 3, 0)],
    1: [("l", 0, 7, 3, 7), ("l", 7, 14, 4, 0), ("l", 21, 14, 5, 0),
        ("l", 35, 14, 6, 0), ("l", 49, 7, 7, 0)],
    2: [("l", 0, 7, 7, 7), ("l", 7, 14, 8, 0), ("l", 21, 14, 9, 0),
        ("l", 35, 14, 10, 0), ("l", 49, 7, 11, 0)],
    3: [("l", 0, 7, 11, 7), ("l", 7, 14, 12, 0), ("l", 21, 14, 13, 0),
        ("l", 35, 14, 14, 0), ("c", 49, 7, 15, 0)],
}


def _host_tables():
    s = (np.arange(_H) + 0.5) / _SCALE - 0.5
    f = np.floor(s)
    w = (s - f).astype(np.float32)
    i0 = np.clip(f, 0, _S - 1).astype(np.int32)
    i1 = np.clip(f + 1, 0, _S - 1).astype(np.int32)
    return i0, i1, (1.0 - w), w


def _sc_body(x_hbm, mask_hbm, i0_hbm, i1_hbm, w0_hbm, w1_hbm, out_hbm,
             b0, b1, b2, b3, mh, mv, i0v, i1v, w0v, w1v,
             si0, si1, si2, si3, so0, so1, so2, so3):
    bufs = [b0, b1, b2, b3]
    sin = [si0, si1, si2, si3]
    sout = [so0, so1, so2, so3]
    wid = lax.axis_index("s") * _NC + lax.axis_index("c")
    base_p = wid * _PPW

    pltpu.sync_copy(i0_hbm, i0v)
    pltpu.sync_copy(i1_hbm, i1v)
    pltpu.sync_copy(w0_hbm, w0v)
    pltpu.sync_copy(w1_hbm, w1v)

    def in_slice(p, cp):
        return x_hbm.at[p, pl.ds(cp * _CH, _CH)]

    def out_slice(p, cp):
        return out_hbm.at[p, pl.ds(cp * _CH, _CH)]

    zero = jnp.zeros((_VL,), jnp.float32)

    def build_mh(p):
        pltpu.sync_copy(mask_hbm.at[p], mv)

        def r_body(r, c):
            ro = r * _S
            for v in range(_VPR):
                sl = pl.ds(v * _VL, _VL)
                g0 = plsc.load_gather(mv, [i0v[sl] + ro])
                g1 = plsc.load_gather(mv, [i1v[sl] + ro])
                mh[pl.ds(r * _H + v * _VL, _VL)] = g0 * w0v[sl] + g1 * w1v[sl]
            return c
        lax.fori_loop(0, _S, r_body, 0)

    def seg_const(buf, r0, n, mrow):
        bits = [mh[pl.ds(mrow * _H + v * _VL, _VL)] for v in range(_VPR)]

        def rb(i, c):
            r = r0 + i
            for v in range(_VPR):
                sl = pl.ds(v * _VL, _VL)
                buf[r, sl] = jnp.where(bits[v] < _MASK_FRAC, zero, buf[r, sl])
            return c
        lax.fori_loop(0, n, rb, 0)

    def seg_lerp(buf, r0, n, mrow, woff):
        m0s = [mh[pl.ds(mrow * _H + v * _VL, _VL)] for v in range(_VPR)]
        dvs = [mh[pl.ds((mrow + 1) * _H + v * _VL, _VL)] - m0s[v]
               for v in range(_VPR)]

        def rb(i, c):
            wy = (i.astype(jnp.float32) + (woff + 0.5)) * (1.0 / _SCALE)
            wyv = jnp.full((_VL,), 0.0, jnp.float32) + wy
            r = r0 + i
            for v in range(_VPR):
                sl = pl.ds(v * _VL, _VL)
                m = m0s[v] + wyv * dvs[v]
                buf[r, sl] = jnp.where(m < _MASK_FRAC, zero, buf[r, sl])
            return c
        lax.fori_loop(0, n, rb, 0)

    pltpu.async_copy(in_slice(base_p, 0), bufs[0], sin[0])
    pltpu.async_copy(in_slice(base_p, 1), bufs[1], sin[1])

    def body(j, carry):
        p = base_p + j
        for i in range(4):
            bn = (i + 2) % 4

            def drain():
                pltpu.make_async_copy(bufs[bn], out_slice(p, 0),
                                      sout[bn]).wait()
            if i < 2:
                pl.when(j > 0)(drain)
            else:
                drain()

            if i < 2:
                pltpu.async_copy(in_slice(p, i + 2), bufs[bn], sin[bn])
            else:
                def refill():
                    pltpu.async_copy(in_slice(p + 1, i - 2), bufs[bn],
                                     sin[bn])
                pl.when(j < _PPW - 1)(refill)

            pltpu.make_async_copy(in_slice(p, i), bufs[i], sin[i]).wait()

            if i == 0:
                build_mh(p)

            for kind, r0, n, mrow, woff in _SEGS[i]:
                if kind == "c":
                    seg_const(bufs[i], r0, n, mrow)
                else:
                    seg_lerp(bufs[i], r0, n, mrow, woff)

            pltpu.async_copy(bufs[i], out_slice(p, i), sout[i])
        return carry

    lax.fori_loop(0, _PPW, body, 0)

    last = base_p + _PPW - 1
    pltpu.make_async_copy(bufs[2], out_slice(last, 2), sout[2]).wait()
    pltpu.make_async_copy(bufs[3], out_slice(last, 3), sout[3]).wait()


@jax.jit
def _run(x, mask):
    B, C, H, W = x.shape
    xp = x.reshape(_NP, _H, _H)
    mp = mask.reshape(_NP, _S * _S)
    i0, i1, w0, w1 = _host_tables()

    mesh = plsc.VectorSubcoreMesh(core_axis_name="c", subcore_axis_name="s",
                                  num_cores=_NC, num_subcores=_NS)
    fn = functools.partial(
        pl.kernel,
        out_type=jax.ShapeDtypeStruct((_NP, _H, _H), jnp.float32),
        mesh=mesh,
        compiler_params=pltpu.CompilerParams(needs_layout_passes=False),
        scratch_types=[
            pltpu.VMEM((_CH, _H), jnp.float32),
            pltpu.VMEM((_CH, _H), jnp.float32),
            pltpu.VMEM((_CH, _H), jnp.float32),
            pltpu.VMEM((_CH, _H), jnp.float32),
            pltpu.VMEM((_S * _H,), jnp.float32),
            pltpu.VMEM((_S * _S,), jnp.float32),
            pltpu.VMEM((_H,), jnp.int32),
            pltpu.VMEM((_H,), jnp.int32),
            pltpu.VMEM((_H,), jnp.float32),
            pltpu.VMEM((_H,), jnp.float32),
        ] + [pltpu.SemaphoreType.DMA] * 8,
    )(_sc_body)
    out = fn(xp, mp, jnp.asarray(i0), jnp.asarray(i1),
             jnp.asarray(w0), jnp.asarray(w1))
    return out.reshape(B, C, H, W)


def kernel(x, mask):
    return _run(x, mask)


# speedup vs baseline: 3.4606x; 1.0042x over previous
import functools

import jax
import jax.numpy as jnp
import numpy as np
from jax import lax
from jax.experimental import pallas as pl
from jax.experimental.pallas import tpu as pltpu
from jax.experimental.pallas import tpu_sc as plsc

_MASK_FRAC = 0.5
_S = 16
_H = 224
_SCALE = _H // _S
_NP = 768
_NC = 2
_NS = 16
_NW = _NC * _NS
_PPW = _NP // _NW
_VL = 16
_VPR = _H // _VL
_CH = 56

_SEGS = {
    0: [("c", 0, 7, 0, 0), ("l", 7, 14, 0, 0), ("l", 21, 14, 1, 0),
        ("l", 35, 14, 2, 0), ("l", 49, 7, 3, 0)],
    1: [("l", 0, 7, 3, 7), ("l", 7, 14, 4, 0), ("l", 21, 14, 5, 0),
        ("l", 35, 14, 6, 0), ("l", 49, 7, 7, 0)],
    2: [("l", 0, 7, 7, 7), ("l", 7, 14, 8, 0), ("l", 21, 14, 9, 0),
        ("l", 35, 14, 10, 0), ("l", 49, 7, 11, 0)],
    3: [("l", 0, 7, 11, 7), ("l", 7, 14, 12, 0), ("l", 21, 14, 13, 0),
        ("l", 35, 14, 14, 0), ("c", 49, 7, 15, 0)],
}


def _host_tables():
    s = (np.arange(_H) + 0.5) / _SCALE - 0.5
    f = np.floor(s)
    w = (s - f).astype(np.float32)
    i0 = np.clip(f, 0, _S - 1).astype(np.int32)
    i1 = np.clip(f + 1, 0, _S - 1).astype(np.int32)
    return i0, i1, (1.0 - w), w


def _sc_body(x_hbm, mask_hbm, i0_hbm, i1_hbm, w0_hbm, w1_hbm, out_hbm,
             b0, b1, b2, b3, mh, mv, i0v, i1v, w0v, w1v,
             si0, si1, si2, si3, so0, so1, so2, so3):
    bufs = [b0, b1, b2, b3]
    sin = [si0, si1, si2, si3]
    sout = [so0, so1, so2, so3]
    wid = lax.axis_index("s") * _NC + lax.axis_index("c")
    base_p = wid * _PPW

    pltpu.sync_copy(i0_hbm, i0v)
    pltpu.sync_copy(i1_hbm, i1v)
    pltpu.sync_copy(w0_hbm, w0v)
    pltpu.sync_copy(w1_hbm, w1v)

    def in_slice(p, cp):
        return x_hbm.at[p, pl.ds(cp * _CH, _CH)]

    def out_slice(p, cp):
        return out_hbm.at[p, pl.ds(cp * _CH, _CH)]

    zero = jnp.zeros((_VL,), jnp.float32)

    pltpu.sync_copy(mask_hbm.at[wid], mv)

    i0s = [i0v[pl.ds(v * _VL, _VL)] for v in range(_VPR)]
    i1s = [i1v[pl.ds(v * _VL, _VL)] for v in range(_VPR)]
    w0s = [w0v[pl.ds(v * _VL, _VL)] for v in range(_VPR)]
    w1s = [w1v[pl.ds(v * _VL, _VL)] for v in range(_VPR)]

    def build_mh(j):
        mbase = j * (_S * _S)

        def r_body(r, c):
            ro = mbase + r * _S
            for v in range(_VPR):
                g0 = plsc.load_gather(mv, [i0s[v] + ro])
                g1 = plsc.load_gather(mv, [i1s[v] + ro])
                mh[pl.ds(r * _H + v * _VL, _VL)] = g0 * w0s[v] + g1 * w1s[v]
            return c
        lax.fori_loop(0, _S, r_body, 0)

    def seg_const(buf, r0, n, mrow):
        sel = [mh[pl.ds(mrow * _H + v * _VL, _VL)] < _MASK_FRAC
               for v in range(_VPR)]

        def one_row(r):
            for v in range(_VPR):
                sl = pl.ds(v * _VL, _VL)
                buf[r, sl] = jnp.where(sel[v], zero, buf[r, sl])

        def rb(i, c):
            r = r0 + 2 * i
            one_row(r)
            one_row(r + 1)
            return c
        lax.fori_loop(0, n // 2, rb, 0)
        if n % 2:
            one_row(r0 + n - 1)

    def seg_lerp(buf, r0, n, mrow, woff):
        m0s = [mh[pl.ds(mrow * _H + v * _VL, _VL)] for v in range(_VPR)]
        dvs = [mh[pl.ds((mrow + 1) * _H + v * _VL, _VL)] - m0s[v]
               for v in range(_VPR)]

        def one_row(r, wy):
            wyv = jnp.full((_VL,), 0.0, jnp.float32) + wy
            for v in range(_VPR):
                sl = pl.ds(v * _VL, _VL)
                m = m0s[v] + wyv * dvs[v]
                buf[r, sl] = jnp.where(m < _MASK_FRAC, zero, buf[r, sl])

        def rb(i, c):
            i2 = 2 * i
            wy = (i2.astype(jnp.float32) + (woff + 0.5)) * (1.0 / _SCALE)
            r = r0 + i2
            one_row(r, wy)
            one_row(r + 1, wy + 1.0 / _SCALE)
            return c
        lax.fori_loop(0, n // 2, rb, 0)
        if n % 2:
            one_row(r0 + n - 1, (n - 1 + woff + 0.5) * (1.0 / _SCALE))

    pltpu.async_copy(in_slice(base_p, 0), bufs[0], sin[0])
    pltpu.async_copy(in_slice(base_p, 1), bufs[1], sin[1])

    def body(j, carry):
        p = base_p + j
        for i in range(4):
            bn = (i + 2) % 4

            def drain():
                pltpu.make_async_copy(bufs[bn], out_slice(p, 0),
                                      sout[bn]).wait()
            if i < 2:
                pl.when(j > 0)(drain)
            else:
                drain()

            if i < 2:
                pltpu.async_copy(in_slice(p, i + 2), bufs[bn], sin[bn])
            else:
                def refill():
                    pltpu.async_copy(in_slice(p + 1, i - 2), bufs[bn],
                                     sin[bn])
                pl.when(j < _PPW - 1)(refill)

            pltpu.make_async_copy(in_slice(p, i), bufs[i], sin[i]).wait()

            if i == 0:
                build_mh(j)

            for kind, r0, n, mrow, woff in _SEGS[i]:
                if kind == "c":
                    seg_const(bufs[i], r0, n, mrow)
                else:
                    seg_lerp(bufs[i], r0, n, mrow, woff)

            pltpu.async_copy(bufs[i], out_slice(p, i), sout[i])
        return carry

    lax.fori_loop(0, _PPW, body, 0)

    last = base_p + _PPW - 1
    pltpu.make_async_copy(bufs[2], out_slice(last, 2), sout[2]).wait()
    pltpu.make_async_copy(bufs[3], out_slice(last, 3), sout[3]).wait()


@jax.jit
def _run(x, mask):
    B, C, H, W = x.shape
    xp = x.reshape(_NP, _H, _H)
    mp = mask.reshape(_NW, _PPW * _S * _S)
    i0, i1, w0, w1 = _host_tables()

    mesh = plsc.VectorSubcoreMesh(core_axis_name="c", subcore_axis_name="s",
                                  num_cores=_NC, num_subcores=_NS)
    fn = functools.partial(
        pl.kernel,
        out_type=jax.ShapeDtypeStruct((_NP, _H, _H), jnp.float32),
        mesh=mesh,
        compiler_params=pltpu.CompilerParams(needs_layout_passes=False),
        scratch_types=[
            pltpu.VMEM((_CH, _H), jnp.float32),
            pltpu.VMEM((_CH, _H), jnp.float32),
            pltpu.VMEM((_CH, _H), jnp.float32),
            pltpu.VMEM((_CH, _H), jnp.float32),
            pltpu.VMEM((_S * _H,), jnp.float32),
            pltpu.VMEM((_PPW * _S * _S,), jnp.float32),
            pltpu.VMEM((_H,), jnp.int32),
            pltpu.VMEM((_H,), jnp.int32),
            pltpu.VMEM((_H,), jnp.float32),
            pltpu.VMEM((_H,), jnp.float32),
        ] + [pltpu.SemaphoreType.DMA] * 8,
    )(_sc_body)
    out = fn(xp, mp, jnp.asarray(i0), jnp.asarray(i1),
             jnp.asarray(w0), jnp.asarray(w1))
    return out.reshape(B, C, H, W)


def kernel(x, mask):
    return _run(x, mask)
